# baseline (speedup 1.0000x reference)
import jax
import jax.numpy as jnp
from jax.experimental import pallas as pl
from jax.experimental.pallas import tpu as pltpu

_BN_EPS = 1e-5


def _pick_tile(m, cap=2048):
    if m <= cap:
        return m
    t = (cap // 128) * 128
    while t >= 128:
        if m % t == 0:
            return t
        t -= 128
    return m


def kernel(g, x, wg, gamma_g, beta_g, wx, gamma_x, beta_x, wpsi,
           gamma_p, beta_p):
    N, F_l, H, W = g.shape
    _, F_g, _, _ = x.shape
    F_int = wg.shape[0]
    M = H * W
    TILE = _pick_tile(M)
    T = M // TILE
    inv = 1.0 / (N * M)

    g3 = g.reshape(N, F_l, M)
    x3 = x.reshape(N, F_g, M)
    bn1 = jnp.stack([gamma_g, beta_g, gamma_x, beta_x], axis=1)
    bnp = jnp.stack([gamma_p, beta_p], axis=1)

    def body(g_ref, x_ref, wg_ref, wx_ref, bn1_ref, wpsi_ref, bnp_ref,
             o_ref, y_s, psi_s, st_s, ps_s):
        j = pl.program_id(0)
        t = pl.program_id(1)

        @pl.when(jnp.logical_and(j == 0, t == 0))
        def _init():
            st_s[...] = jnp.zeros_like(st_s)
            ps_s[...] = jnp.zeros_like(ps_s)

        @pl.when(j < N)
        def _phase_a():
            n = j
            gb = g_ref[0].astype(jnp.bfloat16)
            xb = x_ref[0].astype(jnp.bfloat16)
            u = jnp.dot(wg_ref[...].astype(jnp.bfloat16), gb,
                        preferred_element_type=jnp.float32)
            v = jnp.dot(wx_ref[...].astype(jnp.bfloat16), xb,
                        preferred_element_type=jnp.float32)
            y_s[n, :F_int, pl.ds(t * TILE, TILE)] = u.astype(jnp.bfloat16)
            y_s[n, F_int:, pl.ds(t * TILE, TILE)] = v.astype(jnp.bfloat16)
            st_s[...] += jnp.concatenate(
                [jnp.sum(u, axis=1, keepdims=True),
                 jnp.sum(u * u, axis=1, keepdims=True),
                 jnp.sum(v, axis=1, keepdims=True),
                 jnp.sum(v * v, axis=1, keepdims=True)], axis=1)

        @pl.when(jnp.logical_and(j >= N, j < 2 * N))
        def _phase_b():
            n = j - N
            s = st_s[...]
            mu = s[:, 0:1] * inv
            vu = s[:, 1:2] * inv - mu * mu
            su = bn1_ref[:, 0:1] * jax.lax.rsqrt(vu + _BN_EPS)
            hu = bn1_ref[:, 1:2] - mu * su
            mv = s[:, 2:3] * inv
            vv = s[:, 3:4] * inv - mv * mv
            sv = bn1_ref[:, 2:3] * jax.lax.rsqrt(vv + _BN_EPS)
            hv = bn1_ref[:, 3:4] - mv * sv
            u = y_s[n, :F_int, pl.ds(t * TILE, TILE)]
            v = y_s[n, F_int:, pl.ds(t * TILE, TILE)]
            a = jnp.maximum(u * su + v * sv + (hu + hv), 0.0)
            p = jnp.dot(wpsi_ref[...], a,
                        preferred_element_type=jnp.float32)
            psi_s[n, :, pl.ds(t * TILE, TILE)] = p
            ps_s[...] += jnp.concatenate(
                [jnp.sum(p, axis=1, keepdims=True),
                 jnp.sum(p * p, axis=1, keepdims=True)], axis=1)

        @pl.when(j >= 2 * N)
        def _phase_c():
            n = j - 2 * N
            s = ps_s[...]
            m = s[:, 0:1] * inv
            var = s[:, 1:2] * inv - m * m
            sc = bnp_ref[:, 0:1] * jax.lax.rsqrt(var + _BN_EPS)
            sh = bnp_ref[:, 1:2] - m * sc
            z = psi_s[n, :, pl.ds(t * TILE, TILE)] * sc + sh
            gate = 1.0 / (1.0 + jnp.exp(-z))
            o_ref[0] = x_ref[0] * gate

    def vconst(shape):
        return pl.BlockSpec(shape, lambda j, t: (0,) * len(shape))

    def g_idx(j, t):
        hold = j < N
        return (jnp.where(hold, j, N - 1), 0, jnp.where(hold, t, T - 1))

    def x_idx(j, t):
        in_a = j < N
        in_c = j >= 2 * N
        return (jnp.where(in_a, j, jnp.where(in_c, j - 2 * N, N - 1)), 0,
                jnp.where(jnp.logical_or(in_a, in_c), t, T - 1))

    def o_idx(j, t):
        in_c = j >= 2 * N
        return (jnp.where(in_c, j - 2 * N, 0), 0, jnp.where(in_c, t, 0))

    out = pl.pallas_call(
        body,
        out_shape=jax.ShapeDtypeStruct((N, F_g, M), jnp.float32),
        grid=(3 * N, T),
        in_specs=[
            pl.BlockSpec((1, F_l, TILE), g_idx),
            pl.BlockSpec((1, F_g, TILE), x_idx),
            vconst((F_int, F_l)),
            vconst((F_int, F_g)),
            vconst((F_int, 4)),
            vconst((1, F_int)),
            vconst((1, 2)),
        ],
        out_specs=pl.BlockSpec((1, F_g, TILE), o_idx),
        scratch_shapes=[
            pltpu.VMEM((N, 2 * F_int, M), jnp.bfloat16),
            pltpu.VMEM((N, 1, M), jnp.float32),
            pltpu.VMEM((F_int, 4), jnp.float32),
            pltpu.VMEM((1, 2), jnp.float32),
        ],
        compiler_params=pltpu.CompilerParams(
            dimension_semantics=("arbitrary", "arbitrary")),
    )(g3, x3, wg, wx, bn1, wpsi, bnp)

    return out.reshape(N, F_g, H, W)
